# baseline (speedup 1.0000x reference)
import jax
import jax.numpy as jnp
from jax import lax
from jax.experimental import pallas as pl
from jax.experimental.pallas import tpu as pltpu
from jax.experimental.pallas import tpu_sc as plsc

N = 10000
E = 320000
F = 128
NC = 2
NS = 16
NW = NC * NS
N_PAD = 10240
ROWS_PER_SUB = N_PAD // NS
CH = 96
NCH = 108
PHASES = 2
PHCH = NCH // PHASES
PHP = PHCH // 2
PHE = PHCH * CH
E_PAD = NW * NCH * CH
L = 16


def _agg_body(with_deg):
    def body(*refs):
        if with_deg:
            (feat_hbm, src_hbm, dst_hbm, zeros_hbm,
             out0_hbm, out1_hbm, deg_hbm,
             src_all, dst_all, src_va, dst_va, src_vb, dst_vb,
             rows_a, rows_b, hist, acc, sema, semb) = refs
        else:
            (feat_hbm, src_hbm, dst_hbm, zeros_hbm,
             out0_hbm, out1_hbm,
             src_all, dst_all, src_va, dst_va, src_vb, dst_vb,
             rows_a, rows_b, acc, sema, semb) = refs
        c = lax.axis_index("c")
        s = lax.axis_index("s")
        wid = s * NC + c
        r0 = s * ROWS_PER_SUB
        ebase = wid * NCH * CH

        pltpu.sync_copy(zeros_hbm.at[pl.ds(r0, ROWS_PER_SUB)],
                        acc.at[pl.ds(r0, ROWS_PER_SUB)])
        if with_deg:
            def zstep(i, carry):
                hist[pl.ds(i * L, L)] = jnp.zeros((L,), jnp.float32)
                return carry
            lax.fori_loop(0, N_PAD // L, zstep, 0)
        plsc.subcore_barrier()

        ones = jnp.ones((L,), jnp.float32)

        def move_idx(chunk, sv, dv):
            off = chunk * CH
            for k in range(CH // L):
                sv[pl.ds(k * L, L)] = src_all[pl.ds(off + k * L, L)]
                dv[pl.ds(k * L, L)] = dst_all[pl.ds(off + k * L, L)]

        def do_hist(dv):
            for k in range(CH // L):
                plsc.addupdate_scatter(hist, [dv[pl.ds(k * L, L)]], ones)

        def gather(sv, rows, sem):
            return pltpu.async_copy(feat_hbm.at[sv], rows, sem)

        def scatter(rows, dv):
            pltpu.sync_copy(rows, acc.at[dv], add=True)

        for ph in range(PHASES):
            pltpu.sync_copy(src_hbm.at[pl.ds(ebase + ph * PHE, PHE)],
                            src_all)
            pltpu.sync_copy(dst_hbm.at[pl.ds(ebase + ph * PHE, PHE)],
                            dst_all)
            def pair(i, carry):
                i2 = 2 * i
                move_idx(i2, src_va, dst_va)
                da = gather(src_va, rows_a, sema)
                move_idx(i2 + 1, src_vb, dst_vb)
                db = gather(src_vb, rows_b, semb)
                da.wait()
                scatter(rows_a, dst_va)
                if with_deg:
                    do_hist(dst_va)
                db.wait()
                scatter(rows_b, dst_vb)
                if with_deg:
                    do_hist(dst_vb)
                return carry

            lax.fori_loop(0, PHP, pair, 0)
        plsc.subcore_barrier()

        @pl.when(c == 0)
        def _():
            pltpu.sync_copy(acc.at[pl.ds(r0, ROWS_PER_SUB)],
                            out0_hbm.at[pl.ds(r0, ROWS_PER_SUB)])

        @pl.when(c == 1)
        def _():
            pltpu.sync_copy(acc.at[pl.ds(r0, ROWS_PER_SUB)],
                            out1_hbm.at[pl.ds(r0, ROWS_PER_SUB)])

        if with_deg:
            pltpu.sync_copy(hist, deg_hbm.at[wid])
    return body


def _make_agg(with_deg):
    scratch = [
        pltpu.VMEM((PHE,), jnp.int32),
        pltpu.VMEM((PHE,), jnp.int32),
        pltpu.VMEM((CH,), jnp.int32),
        pltpu.VMEM((CH,), jnp.int32),
        pltpu.VMEM((CH,), jnp.int32),
        pltpu.VMEM((CH,), jnp.int32),
        pltpu.VMEM((CH, F), jnp.float32),
        pltpu.VMEM((CH, F), jnp.float32),
    ]
    out_type = [jax.ShapeDtypeStruct((N_PAD, F), jnp.float32),
                jax.ShapeDtypeStruct((N_PAD, F), jnp.float32)]
    if with_deg:
        scratch += [pltpu.VMEM((N_PAD,), jnp.float32)]
        out_type += [jax.ShapeDtypeStruct((NW, N_PAD), jnp.float32)]
    scratch += [pltpu.VMEM_SHARED((N_PAD, F), jnp.float32),
                pltpu.SemaphoreType.DMA,
                pltpu.SemaphoreType.DMA]
    return pl.kernel(
        _agg_body(with_deg),
        out_type=tuple(out_type),
        mesh=plsc.VectorSubcoreMesh(core_axis_name="c", subcore_axis_name="s"),
        scratch_types=scratch,
        compiler_params=pltpu.CompilerParams(needs_layout_passes=False),
        name="sage_agg_deg" if with_deg else "sage_agg",
    )


_agg_deg_call = _make_agg(True)
_agg_call = _make_agg(False)

BLK = 1280


def _combine_body(relu):
    def body(x_ref, p0_ref, p1_ref, dp_ref, ws_ref, wn_ref, b_ref, out_ref):
        agg = p0_ref[...] + p1_ref[...]
        deg = jnp.sum(dp_ref[...], axis=0).reshape(BLK, 1)
        mean = agg * (1.0 / jnp.maximum(deg, 1.0))
        y = (jnp.dot(x_ref[...], ws_ref[...],
                     preferred_element_type=jnp.float32)
             + jnp.dot(mean, wn_ref[...], preferred_element_type=jnp.float32)
             + b_ref[...])
        out_ref[...] = jnp.maximum(y, 0.0) if relu else y
    return body


def _combine(x, p0, p1, degparts, Ws, Wn, b, relu):
    return pl.pallas_call(
        _combine_body(relu),
        out_shape=jax.ShapeDtypeStruct((N_PAD, F), jnp.float32),
        grid=(N_PAD // BLK,),
        in_specs=[
            pl.BlockSpec((BLK, F), lambda i: (i, 0)),
            pl.BlockSpec((BLK, F), lambda i: (i, 0)),
            pl.BlockSpec((BLK, F), lambda i: (i, 0)),
            pl.BlockSpec((NW, BLK), lambda i: (0, i)),
            pl.BlockSpec((F, F), lambda i: (0, 0)),
            pl.BlockSpec((F, F), lambda i: (0, 0)),
            pl.BlockSpec((1, F), lambda i: (0, 0)),
        ],
        out_specs=pl.BlockSpec((BLK, F), lambda i: (i, 0)),
    )(x, p0, p1, degparts, Ws, Wn, b.reshape(1, F))


def kernel(x, edge_index, W_self1, W_neigh1, b1, W_self2, W_neigh2, b2):
    src = edge_index[0]
    dst = edge_index[1]
    npad = E_PAD - E
    src_p = jnp.concatenate([src, jnp.zeros((npad,), jnp.int32)])
    pad_dst = N + jnp.arange(npad, dtype=jnp.int32) % (N_PAD - N)
    dst_p = jnp.concatenate([dst, pad_dst])
    x_pad = jnp.pad(x, ((0, N_PAD - N), (0, 0)))
    zeros_hbm = jnp.zeros((N_PAD, F), jnp.float32)

    p0, p1, degparts = _agg_deg_call(x_pad, src_p, dst_p, zeros_hbm)
    h = _combine(x_pad, p0, p1, degparts, W_self1, W_neigh1, b1, relu=True)
    q0, q1 = _agg_call(h, src_p, dst_p, zeros_hbm)
    out = _combine(h, q0, q1, degparts, W_self2, W_neigh2, b2, relu=False)
    return out[:N]


# speedup vs baseline: 3.6771x; 3.6771x over previous
import jax
import jax.numpy as jnp
from jax import lax
from jax.experimental import pallas as pl
from jax.experimental.pallas import tpu as pltpu
from jax.experimental.pallas import tpu_sc as plsc

N = 10000
E = 320000
F = 128
NC = 2
NS = 16
NW = NC * NS
N_PAD = 10240
ROWS_PER_SUB = N_PAD // NS
EPW = E // NW
CHUNK = 128
NFULL = EPW // CHUNK
TAIL = EPW - NFULL * CHUNK
L = 16


def _agg_body(with_deg):
    def body(*refs):
        if with_deg:
            (feat_hbm, src_hbm, dst_hbm, zeros_hbm,
             out0_hbm, out1_hbm, deg_hbm,
             src_all, dst_all, src_v, dst_v, src_t, dst_t, rows_v,
             hist, acc, sem) = refs
        else:
            (feat_hbm, src_hbm, dst_hbm, zeros_hbm,
             out0_hbm, out1_hbm,
             src_all, dst_all, src_v, dst_v, src_t, dst_t, rows_v,
             acc, sem) = refs
        c = lax.axis_index("c")
        s = lax.axis_index("s")
        wid = s * NC + c
        r0 = s * ROWS_PER_SUB
        pltpu.sync_copy(zeros_hbm.at[pl.ds(r0, ROWS_PER_SUB)],
                        acc.at[pl.ds(r0, ROWS_PER_SUB)])
        pltpu.sync_copy(src_hbm.at[pl.ds(wid * EPW, EPW)], src_all)
        pltpu.sync_copy(dst_hbm.at[pl.ds(wid * EPW, EPW)], dst_all)
        if with_deg:
            def zstep(i, carry):
                hist[pl.ds(i * L, L)] = jnp.zeros((L,), jnp.float32)
                return carry
            lax.fori_loop(0, N_PAD // L, zstep, 0)
        plsc.subcore_barrier()

        def step(i, carry):
            off = i * CHUNK
            for k in range(CHUNK // L):
                src_v[pl.ds(k * L, L)] = src_all[pl.ds(off + k * L, L)]
                dst_v[pl.ds(k * L, L)] = dst_all[pl.ds(off + k * L, L)]
            pltpu.async_copy(feat_hbm.at[src_v], rows_v, sem).wait()
            pltpu.sync_copy(rows_v, acc.at[dst_v], add=True)
            if with_deg:
                for j in range(CHUNK // L):
                    idx = dst_v[pl.ds(j * L, L)]
                    plsc.addupdate_scatter(hist, [idx],
                                           jnp.ones((L,), jnp.float32))
            return carry

        lax.fori_loop(0, NFULL, step, 0)
        if TAIL:
            off = NFULL * CHUNK
            src_t[...] = src_all[pl.ds(off, TAIL)]
            dst_t[...] = dst_all[pl.ds(off, TAIL)]
            pltpu.async_copy(feat_hbm.at[src_t], rows_v.at[pl.ds(0, TAIL)],
                             sem).wait()
            pltpu.sync_copy(rows_v.at[pl.ds(0, TAIL)], acc.at[dst_t],
                            add=True)
            if with_deg:
                for j in range(TAIL // L):
                    idx = dst_t[pl.ds(j * L, L)]
                    plsc.addupdate_scatter(hist, [idx],
                                           jnp.ones((L,), jnp.float32))
        plsc.subcore_barrier()

        @pl.when(c == 0)
        def _():
            pltpu.sync_copy(acc.at[pl.ds(r0, ROWS_PER_SUB)],
                            out0_hbm.at[pl.ds(r0, ROWS_PER_SUB)])

        @pl.when(c == 1)
        def _():
            pltpu.sync_copy(acc.at[pl.ds(r0, ROWS_PER_SUB)],
                            out1_hbm.at[pl.ds(r0, ROWS_PER_SUB)])

        if with_deg:
            pltpu.sync_copy(hist, deg_hbm.at[wid])
    return body


def _make_agg(with_deg):
    scratch = [
        pltpu.VMEM((EPW,), jnp.int32),
        pltpu.VMEM((EPW,), jnp.int32),
        pltpu.VMEM((CHUNK,), jnp.int32),
        pltpu.VMEM((CHUNK,), jnp.int32),
        pltpu.VMEM((TAIL,), jnp.int32),
        pltpu.VMEM((TAIL,), jnp.int32),
        pltpu.VMEM((CHUNK, F), jnp.float32),
    ]
    out_type = [jax.ShapeDtypeStruct((N_PAD, F), jnp.float32),
                jax.ShapeDtypeStruct((N_PAD, F), jnp.float32)]
    if with_deg:
        scratch += [pltpu.VMEM((N_PAD,), jnp.float32)]
        out_type += [jax.ShapeDtypeStruct((NW, N_PAD), jnp.float32)]
    scratch += [pltpu.VMEM_SHARED((N_PAD, F), jnp.float32),
                pltpu.SemaphoreType.DMA]
    return pl.kernel(
        _agg_body(with_deg),
        out_type=tuple(out_type),
        mesh=plsc.VectorSubcoreMesh(core_axis_name="c", subcore_axis_name="s"),
        scratch_types=scratch,
        compiler_params=pltpu.CompilerParams(needs_layout_passes=False),
        name="sage_agg_deg" if with_deg else "sage_agg",
    )


_agg_deg_call = _make_agg(True)
_agg_call = _make_agg(False)

BLK = 1280


def _combine_body(relu):
    def body(x_ref, p0_ref, p1_ref, dp_ref, ws_ref, wn_ref, b_ref, out_ref):
        agg = p0_ref[...] + p1_ref[...]
        deg = jnp.sum(dp_ref[...], axis=0).reshape(BLK, 1)
        mean = agg * (1.0 / jnp.maximum(deg, 1.0))
        y = (jnp.dot(x_ref[...], ws_ref[...],
                     preferred_element_type=jnp.float32)
             + jnp.dot(mean, wn_ref[...], preferred_element_type=jnp.float32)
             + b_ref[...])
        out_ref[...] = jnp.maximum(y, 0.0) if relu else y
    return body


def _combine(x, p0, p1, degparts, Ws, Wn, b, relu):
    return pl.pallas_call(
        _combine_body(relu),
        out_shape=jax.ShapeDtypeStruct((N_PAD, F), jnp.float32),
        grid=(N_PAD // BLK,),
        in_specs=[
            pl.BlockSpec((BLK, F), lambda i: (i, 0)),
            pl.BlockSpec((BLK, F), lambda i: (i, 0)),
            pl.BlockSpec((BLK, F), lambda i: (i, 0)),
            pl.BlockSpec((NW, BLK), lambda i: (0, i)),
            pl.BlockSpec((F, F), lambda i: (0, 0)),
            pl.BlockSpec((F, F), lambda i: (0, 0)),
            pl.BlockSpec((1, F), lambda i: (0, 0)),
        ],
        out_specs=pl.BlockSpec((BLK, F), lambda i: (i, 0)),
    )(x, p0, p1, degparts, Ws, Wn, b.reshape(1, F))


def kernel(x, edge_index, W_self1, W_neigh1, b1, W_self2, W_neigh2, b2):
    src = edge_index[0]
    dst = edge_index[1]
    x_pad = jnp.pad(x, ((0, N_PAD - N), (0, 0)))
    zeros_hbm = jnp.zeros((N_PAD, F), jnp.float32)

    p0, p1, degparts = _agg_deg_call(x_pad, src, dst, zeros_hbm)
    h = _combine(x_pad, p0, p1, degparts, W_self1, W_neigh1, b1, relu=True)
    q0, q1 = _agg_call(h, src, dst, zeros_hbm)
    out = _combine(h, q0, q1, degparts, W_self2, W_neigh2, b2, relu=False)
    return out[:N]


# speedup vs baseline: 3.7398x; 1.0170x over previous
import jax
import jax.numpy as jnp
from jax import lax
from jax.experimental import pallas as pl
from jax.experimental.pallas import tpu as pltpu
from jax.experimental.pallas import tpu_sc as plsc

N = 10000
E = 320000
F = 128
NC = 2
NS = 16
NW = NC * NS
N_PAD = 10240
ROWS_PER_SUB = N_PAD // NS
EPW = E // NW
CHUNK = 128
NFULL = EPW // CHUNK
TAIL = EPW - NFULL * CHUNK
L = 16


def _agg_body(with_deg):
    def body(*refs):
        if with_deg:
            (feat_hbm, src_hbm, dst_hbm, zeros_hbm,
             out0_hbm, out1_hbm, deg_hbm,
             src_all, dst_all, src_v, dst_v, src_t, dst_t, rows_v,
             hist, acc, sem) = refs
        else:
            (feat_hbm, src_hbm, dst_hbm, zeros_hbm,
             out0_hbm, out1_hbm,
             src_all, dst_all, src_v, dst_v, src_t, dst_t, rows_v,
             acc, sem) = refs
        c = lax.axis_index("c")
        s = lax.axis_index("s")
        wid = s * NC + c
        r0 = s * ROWS_PER_SUB
        pltpu.sync_copy(zeros_hbm.at[pl.ds(r0, ROWS_PER_SUB)],
                        acc.at[pl.ds(r0, ROWS_PER_SUB)])
        pltpu.sync_copy(src_hbm.at[pl.ds(wid * EPW, EPW)], src_all)
        pltpu.sync_copy(dst_hbm.at[pl.ds(wid * EPW, EPW)], dst_all)
        if with_deg:
            def zstep(i, carry):
                hist[pl.ds(i * L, L)] = jnp.zeros((L,), jnp.float32)
                return carry
            lax.fori_loop(0, N_PAD // L, zstep, 0)
        plsc.subcore_barrier()

        def step(i, carry):
            off = i * CHUNK
            for k in range(CHUNK // L):
                src_v[pl.ds(k * L, L)] = src_all[pl.ds(off + k * L, L)]
                dst_v[pl.ds(k * L, L)] = dst_all[pl.ds(off + k * L, L)]
            pltpu.async_copy(feat_hbm.at[src_v], rows_v, sem).wait()
            pltpu.sync_copy(rows_v, acc.at[dst_v], add=True)
            if with_deg:
                for j in range(CHUNK // L):
                    idx = dst_v[pl.ds(j * L, L)]
                    plsc.addupdate_scatter(hist, [idx],
                                           jnp.ones((L,), jnp.float32))
            return carry

        lax.fori_loop(0, NFULL, step, 0)
        if TAIL:
            off = NFULL * CHUNK
            src_t[...] = src_all[pl.ds(off, TAIL)]
            dst_t[...] = dst_all[pl.ds(off, TAIL)]
            pltpu.async_copy(feat_hbm.at[src_t], rows_v.at[pl.ds(0, TAIL)],
                             sem).wait()
            pltpu.sync_copy(rows_v.at[pl.ds(0, TAIL)], acc.at[dst_t],
                            add=True)
            if with_deg:
                for j in range(TAIL // L):
                    idx = dst_t[pl.ds(j * L, L)]
                    plsc.addupdate_scatter(hist, [idx],
                                           jnp.ones((L,), jnp.float32))
        plsc.subcore_barrier()

        @pl.when(c == 0)
        def _():
            pltpu.sync_copy(acc.at[pl.ds(r0, ROWS_PER_SUB)],
                            out0_hbm.at[pl.ds(r0, ROWS_PER_SUB)])

        @pl.when(c == 1)
        def _():
            pltpu.sync_copy(acc.at[pl.ds(r0, ROWS_PER_SUB)],
                            out1_hbm.at[pl.ds(r0, ROWS_PER_SUB)])

        if with_deg:
            pltpu.sync_copy(hist, deg_hbm.at[wid])
    return body


def _make_agg(with_deg):
    scratch = [
        pltpu.VMEM((EPW,), jnp.int32),
        pltpu.VMEM((EPW,), jnp.int32),
        pltpu.VMEM((CHUNK,), jnp.int32),
        pltpu.VMEM((CHUNK,), jnp.int32),
        pltpu.VMEM((TAIL,), jnp.int32),
        pltpu.VMEM((TAIL,), jnp.int32),
        pltpu.VMEM((CHUNK, F), jnp.float32),
    ]
    out_type = [jax.ShapeDtypeStruct((N_PAD, F), jnp.float32),
                jax.ShapeDtypeStruct((N_PAD, F), jnp.float32)]
    if with_deg:
        scratch += [pltpu.VMEM((N_PAD,), jnp.float32)]
        out_type += [jax.ShapeDtypeStruct((NW, N_PAD), jnp.float32)]
    scratch += [pltpu.VMEM_SHARED((N_PAD, F), jnp.float32),
                pltpu.SemaphoreType.DMA]
    return pl.kernel(
        _agg_body(with_deg),
        out_type=tuple(out_type),
        mesh=plsc.VectorSubcoreMesh(core_axis_name="c", subcore_axis_name="s"),
        scratch_types=scratch,
        compiler_params=pltpu.CompilerParams(needs_layout_passes=False),
        name="sage_agg_deg" if with_deg else "sage_agg",
    )


_agg_deg_call = _make_agg(True)
_agg_call = _make_agg(False)

BLK = 1280


def _combine_body(relu):
    def body(x_ref, p0_ref, p1_ref, dp_ref, ws_ref, wn_ref, b_ref, out_ref):
        agg = p0_ref[...] + p1_ref[...]
        deg = jnp.sum(dp_ref[...], axis=0).reshape(BLK, 1)
        mean = agg * (1.0 / jnp.maximum(deg, 1.0))
        y = (jnp.dot(x_ref[...], ws_ref[...],
                     preferred_element_type=jnp.float32)
             + jnp.dot(mean, wn_ref[...], preferred_element_type=jnp.float32)
             + b_ref[...])
        out_ref[...] = jnp.maximum(y, 0.0) if relu else y
    return body


def _combine(x, p0, p1, degparts, Ws, Wn, b, relu):
    return pl.pallas_call(
        _combine_body(relu),
        out_shape=jax.ShapeDtypeStruct((N, F), jnp.float32),
        grid=(N_PAD // BLK,),
        in_specs=[
            pl.BlockSpec((BLK, F), lambda i: (i, 0)),
            pl.BlockSpec((BLK, F), lambda i: (i, 0)),
            pl.BlockSpec((BLK, F), lambda i: (i, 0)),
            pl.BlockSpec((NW, BLK), lambda i: (0, i)),
            pl.BlockSpec((F, F), lambda i: (0, 0)),
            pl.BlockSpec((F, F), lambda i: (0, 0)),
            pl.BlockSpec((1, F), lambda i: (0, 0)),
        ],
        out_specs=pl.BlockSpec((BLK, F), lambda i: (i, 0)),
    )(x, p0, p1, degparts, Ws, Wn, b.reshape(1, F))


def kernel(x, edge_index, W_self1, W_neigh1, b1, W_self2, W_neigh2, b2):
    src = edge_index[0]
    dst = edge_index[1]
    zeros_hbm = jnp.zeros((N_PAD, F), jnp.float32)

    p0, p1, degparts = _agg_deg_call(x, src, dst, zeros_hbm)
    h = _combine(x, p0, p1, degparts, W_self1, W_neigh1, b1, relu=True)
    q0, q1 = _agg_call(h, src, dst, zeros_hbm)
    return _combine(h, q0, q1, degparts, W_self2, W_neigh2, b2, relu=False)
